# baseline (speedup 1.0000x reference)
import jax
import jax.numpy as jnp
from jax import lax
from jax.experimental import pallas as pl
from jax.experimental.pallas import tpu as pltpu
from jax.experimental.pallas import tpu_sc as plsc

NUM_CORES = 2
NUM_SUBCORES = 16
NW = NUM_CORES * NUM_SUBCORES
LANES = 16
BATCH = 16384
NX = 2 * BATCH
NV = NX // LANES
DIM = 32
NUM_E = 1000000
NUM_R = 1000
CE = 128
NBUF = 6
NFULL = NUM_E // CE
TAIL_BASE = NFULL * CE
TAILBUF_BASE = NUM_E - 128
BASE_CH = NFULL // NW
EXTRA = NFULL - BASE_CH * NW
RING = 8
STG = LANES * DIM
NDUMP = NW * LANES
SUB = 128
NSUB = (BATCH // NW) // SUB


def _extract_body(xs_s, order, eht, ett, lasth, lastt, hstage, tstage,
                  xsv, ordv, hch, tch, stgh, stgt, coltmp, jtmp, sem, sem2):
    c = lax.axis_index("c")
    s = lax.axis_index("s")
    wid = s * NUM_CORES + c
    lo_chunk = wid * BASE_CH + jnp.minimum(wid, EXTRA)
    nreg = BASE_CH + jnp.where(wid < EXTRA, 1, 0)
    is_last = wid == NW - 1
    nch = nreg + jnp.where(is_last, 1, 0)
    elo = lo_chunk * CE

    pltpu.sync_copy(xs_s, xsv)
    pltpu.sync_copy(order, ordv)

    iota = lax.iota(jnp.int32, LANES)

    def bis(_, st):
        lo, hi = st
        mid = (lo + hi) >> 1
        v0 = xsv[pl.ds(mid * LANES, LANES)][0]
        below = v0 < elo
        return (jnp.where(below, mid, lo), jnp.where(below, hi, mid))

    p0, _ = lax.fori_loop(0, 12, bis, (jnp.int32(0), jnp.int32(NV)))

    def scan_chunk(ci, coladd, P, f):
        cbase = ci * CE

        def wcond(st):
            return st[2]

        def wbody(st):
            P, f, _ = st
            xv = xsv[pl.ds(P * LANES, LANES)]
            jv16 = ordv[pl.ds(P * LANES, LANES)]
            xoff = xv - elo
            m = (xoff >= cbase) & (xoff < cbase + CE)
            cnt = plsc.all_reduce_population_count(m)[0]

            @pl.when(cnt > 0)
            def _fire():
                slot = lax.rem(f, RING)
                sb = slot * STG
                @pl.when(f >= RING - 1)
                def _drain():
                    pltpu.make_async_copy(
                        hstage.at[pl.ds(0, STG)], stgh.at[pl.ds(0, STG)],
                        sem).wait()
                    pltpu.make_async_copy(
                        tstage.at[pl.ds(0, STG)], stgt.at[pl.ds(0, STG)],
                        sem).wait()
                rank = plsc.cumsum(m.astype(jnp.int32)) - 1
                coltmp[...] = jnp.zeros((LANES,), jnp.int32)
                jtmp[...] = NX + wid * LANES + iota
                plsc.store_scatter(coltmp, [rank], xoff - cbase + coladd,
                                   mask=m)
                plsc.store_scatter(jtmp, [rank], jv16, mask=m)
                ctv = coltmp[...]
                jv = jtmp[...]
                for d in range(DIM):
                    dsp = jnp.full((LANES,), d, jnp.int32)
                    hv = plsc.load_gather(hch, [dsp, ctv])
                    tv = plsc.load_gather(tch, [dsp, ctv])
                    plsc.store_scatter(stgh, [sb + iota * DIM + d], hv)
                    plsc.store_scatter(stgt, [sb + iota * DIM + d], tv)
                for k in range(LANES):
                    jk = jv[k]
                    pltpu.async_copy(stgh.at[pl.ds(sb + k * DIM, DIM)],
                                     hstage.at[pl.ds(jk * DIM, DIM)], sem)
                    pltpu.async_copy(stgt.at[pl.ds(sb + k * DIM, DIM)],
                                     tstage.at[pl.ds(jk * DIM, DIM)], sem)

            fn = f + jnp.where(cnt > 0, 1, 0)
            all_below = (xv[LANES - 1] - elo) < cbase + CE
            pn = jnp.minimum(P + 1, NV - 1)
            go = all_below & (pn != P)
            return (jnp.where(all_below, pn, P), fn, go)

        P, f, _ = lax.while_loop(wcond, wbody, (P, f, jnp.bool_(True)))
        return P, f

    def fire_chunk(ci):
        par = lax.rem(ci, NBUF) * CE

        @pl.when(ci < nreg)
        def _reg():
            pltpu.async_copy(
                eht.at[:, pl.ds(elo + ci * CE, CE)],
                hch.at[:, pl.ds(par, CE)], sem2)
            pltpu.async_copy(
                ett.at[:, pl.ds(elo + ci * CE, CE)],
                tch.at[:, pl.ds(par, CE)], sem2)

        @pl.when(is_last & (ci == nreg))
        def _tail():
            pltpu.async_copy(lasth, hch.at[:, pl.ds(par, 128)], sem2)
            pltpu.async_copy(lastt, tch.at[:, pl.ds(par, 128)], sem2)

    for pre in range(NBUF - 1):
        fire_chunk(pre)

    def chunk_fn(ci, st):
        P, f = st
        par = lax.rem(ci, NBUF) * CE
        is_tail = is_last & (ci == nreg)

        @pl.when(jnp.logical_not(is_tail))
        def _dr():
            pltpu.make_async_copy(eht.at[:, pl.ds(0, CE)],
                                  hch.at[:, pl.ds(par, CE)], sem2).wait()
            pltpu.make_async_copy(eht.at[:, pl.ds(0, CE)],
                                  tch.at[:, pl.ds(par, CE)], sem2).wait()

        @pl.when(is_tail)
        def _drt():
            pltpu.make_async_copy(eht.at[:, pl.ds(0, 128)],
                                  hch.at[:, pl.ds(par, 128)], sem2).wait()
            pltpu.make_async_copy(eht.at[:, pl.ds(0, 128)],
                                  tch.at[:, pl.ds(par, 128)], sem2).wait()

        fire_chunk(ci + NBUF - 1)
        coladd = par + jnp.where(is_tail, 64, 0)
        return scan_chunk(ci, coladd, P, f)

    _, f = lax.fori_loop(0, nch, chunk_fn, (p0, jnp.int32(0)))

    for k in range(RING - 1):
        @pl.when(f > k)
        def _final_drain():
            pltpu.make_async_copy(
                hstage.at[pl.ds(0, STG)], stgh.at[pl.ds(0, STG)], sem).wait()
            pltpu.make_async_copy(
                tstage.at[pl.ds(0, STG)], stgt.at[pl.ds(0, STG)], sem).wait()


def _combine_body(rels, hstage, tstage, rf, ri, out,
                  ridx, h1b, t1b, h2b, t2b, r1b, r2b, qt, outv, sem):
    c = lax.axis_index("c")
    s = lax.axis_index("s")
    wid = s * NUM_CORES + c
    base = wid * (BATCH // NW)
    sz = SUB * DIM

    pltpu.sync_copy(rels.at[pl.ds(base, BATCH // NW)], ridx)

    iota = lax.iota(jnp.int32, LANES)

    for sb in range(NSUB):
        s0 = base + sb * SUB
        off = sb * sz
        pltpu.async_copy(hstage.at[pl.ds(s0 * DIM, sz)],
                         h1b.at[pl.ds(off, sz)], sem)
        pltpu.async_copy(tstage.at[pl.ds(s0 * DIM, sz)],
                         t2b.at[pl.ds(off, sz)], sem)
        pltpu.async_copy(hstage.at[pl.ds((s0 + BATCH) * DIM, sz)],
                         h2b.at[pl.ds(off, sz)], sem)
        pltpu.async_copy(tstage.at[pl.ds((s0 + BATCH) * DIM, sz)],
                         t1b.at[pl.ds(off, sz)], sem)
        rsl = ridx.at[pl.ds(sb * SUB, SUB)]
        pltpu.async_copy(rf.at[rsl], r1b.at[pl.ds(sb * SUB, SUB)], sem)
        pltpu.async_copy(ri.at[rsl], r2b.at[pl.ds(sb * SUB, SUB)], sem)

    def sub_fn(sb, s0, off):
        pltpu.make_async_copy(hstage.at[pl.ds(0, sz)],
                              h1b.at[pl.ds(off, sz)], sem).wait()
        pltpu.make_async_copy(hstage.at[pl.ds(0, sz)],
                              t2b.at[pl.ds(off, sz)], sem).wait()
        pltpu.make_async_copy(hstage.at[pl.ds(0, sz)],
                              h2b.at[pl.ds(off, sz)], sem).wait()
        pltpu.make_async_copy(hstage.at[pl.ds(0, sz)],
                              t1b.at[pl.ds(off, sz)], sem).wait()
        pltpu.make_async_copy(rf.at[pl.ds(0, SUB)],
                              r1b.at[pl.ds(sb * SUB, SUB)], sem).wait()
        pltpu.make_async_copy(rf.at[pl.ds(0, SUB)],
                              r2b.at[pl.ds(sb * SUB, SUB)], sem).wait()
        def row_fn(i, carry):
            lo = pl.ds(i * DIM, LANES)
            hi = pl.ds(i * DIM + LANES, LANES)
            rlo = pl.ds(0, LANES)
            rhi = pl.ds(LANES, LANES)
            pa = (h1b[lo] * r1b[i, rlo] * t1b[lo]
                  + h2b[lo] * r2b[i, rlo] * t2b[lo])
            pb = (h1b[hi] * r1b[i, rhi] * t1b[hi]
                  + h2b[hi] * r2b[i, rhi] * t2b[hi])
            q = pa + pb
            g = i // LANES
            l = i - g * LANES
            flat = g * (LANES * LANES) + iota * LANES + l
            plsc.store_scatter(qt, [flat], q)
            return carry

        lax.fori_loop(sb * SUB, (sb + 1) * SUB, row_fn, 0)

    for sb in range(NSUB):
        sub_fn(sb, base + sb * SUB, sb * sz)

    def grp_fn(g, carry):
        gb = g * (LANES * LANES)
        acc = qt[pl.ds(gb, LANES)]
        for dd in range(1, LANES):
            acc = acc + qt[pl.ds(gb + dd * LANES, LANES)]
        outv[pl.ds(g * LANES, LANES)] = acc * 0.5
        return carry

    lax.fori_loop(0, (BATCH // NW) // LANES, grp_fn, 0)

    pltpu.sync_copy(outv, out.at[pl.ds(base, BATCH // NW)])


@jax.jit
def kernel(heads, rels, tails, ent_embs_h, ent_embs_t, rel_embs_f, rel_embs_i):
    heads = heads.astype(jnp.int32)
    rels = rels.astype(jnp.int32)
    tails = tails.astype(jnp.int32)

    xs = jnp.concatenate([heads, tails])
    xs_s, order = lax.sort((xs, jnp.arange(NX, dtype=jnp.int32)), num_keys=1)
    eht = ent_embs_h.T
    ett = ent_embs_t.T
    lasth = lax.slice(eht, (0, TAILBUF_BASE), (DIM, NUM_E))
    lastt = lax.slice(ett, (0, TAILBUF_BASE), (DIM, NUM_E))

    mesh = plsc.VectorSubcoreMesh(
        core_axis_name="c", subcore_axis_name="s",
        num_cores=NUM_CORES, num_subcores=NUM_SUBCORES)

    extract = pl.kernel(
        _extract_body,
        out_type=(
            jax.ShapeDtypeStruct(((NX + NDUMP) * DIM,), jnp.float32),
            jax.ShapeDtypeStruct(((NX + NDUMP) * DIM,), jnp.float32),
        ),
        mesh=mesh,
        scratch_types=[
            pltpu.VMEM((NX,), jnp.int32),
            pltpu.VMEM((NX,), jnp.int32),
            pltpu.VMEM((DIM, NBUF * CE), jnp.float32),
            pltpu.VMEM((DIM, NBUF * CE), jnp.float32),
            pltpu.VMEM((RING * STG,), jnp.float32),
            pltpu.VMEM((RING * STG,), jnp.float32),
            pltpu.VMEM((LANES,), jnp.int32),
            pltpu.VMEM((LANES,), jnp.int32),
            pltpu.SemaphoreType.DMA,
            pltpu.SemaphoreType.DMA,
        ],
        compiler_params=pltpu.CompilerParams(needs_layout_passes=False),
        name="simple_extract_sc",
    )
    hstage, tstage = extract(xs_s, order, eht, ett, lasth, lastt)

    combine = pl.kernel(
        _combine_body,
        out_type=jax.ShapeDtypeStruct((BATCH,), jnp.float32),
        mesh=mesh,
        scratch_types=[
            pltpu.VMEM((BATCH // NW,), jnp.int32),
            pltpu.VMEM((NSUB * SUB * DIM,), jnp.float32),
            pltpu.VMEM((NSUB * SUB * DIM,), jnp.float32),
            pltpu.VMEM((NSUB * SUB * DIM,), jnp.float32),
            pltpu.VMEM((NSUB * SUB * DIM,), jnp.float32),
            pltpu.VMEM((NSUB * SUB, DIM), jnp.float32),
            pltpu.VMEM((NSUB * SUB, DIM), jnp.float32),
            pltpu.VMEM(((BATCH // NW) * LANES,), jnp.float32),
            pltpu.VMEM((BATCH // NW,), jnp.float32),
            pltpu.SemaphoreType.DMA,
        ],
        compiler_params=pltpu.CompilerParams(
            needs_layout_passes=False, use_tc_tiling_on_sc=False),
        name="simple_combine_sc",
    )
    return combine(rels, hstage, tstage, rel_embs_f, rel_embs_i)


# speedup vs baseline: 1.4077x; 1.4077x over previous
import jax
import jax.numpy as jnp
from jax import lax
from jax.experimental import pallas as pl
from jax.experimental.pallas import tpu as pltpu
from jax.experimental.pallas import tpu_sc as plsc

NUM_CORES = 2
NUM_SUBCORES = 16
NW = NUM_CORES * NUM_SUBCORES
LANES = 16
BATCH = 16384
NX = 2 * BATCH
NV = NX // LANES
DIM = 32
NUM_E = 1000000
NUM_R = 1000
CE = 256
NBUF = 3
NFULL = NUM_E // CE
TAIL_BASE = NFULL * CE
TAILBUF_BASE = NUM_E - 128
BASE_CH = NFULL // NW
EXTRA = NFULL - BASE_CH * NW
RING = 8
STG = LANES * DIM
NDUMP = NW * LANES
SUB = 128
NSUB = (BATCH // NW) // SUB


def _extract_body(xs_s, order, eht, ett, lasth, lastt, hstage, tstage,
                  xsv, ordv, hch, tch, stgh, stgt, coltmp, jtmp,
                  sem, sem2, sem3):
    c = lax.axis_index("c")
    s = lax.axis_index("s")
    wid = s * NUM_CORES + c
    lo_chunk = wid * BASE_CH + jnp.minimum(wid, EXTRA)
    nreg = BASE_CH + jnp.where(wid < EXTRA, 1, 0)
    is_last = wid == NW - 1
    nch = nreg + jnp.where(is_last, 1, 0)
    elo = lo_chunk * CE

    pltpu.async_copy(xs_s, xsv, sem3)
    pltpu.async_copy(order, ordv, sem3)

    iota = lax.iota(jnp.int32, LANES)

    def bis(_, st):
        lo, hi = st
        mid = (lo + hi) >> 1
        v0 = xsv[pl.ds(mid * LANES, LANES)][0]
        below = v0 < elo
        return (jnp.where(below, mid, lo), jnp.where(below, hi, mid))

    def scan_chunk(ci, coladd, P, f):
        cbase = ci * CE

        def wcond(st):
            return st[2]

        def wbody(st):
            P, f, _ = st
            xv = xsv[pl.ds(P * LANES, LANES)]
            jv16 = ordv[pl.ds(P * LANES, LANES)]
            xoff = xv - elo
            m = (xoff >= cbase) & (xoff < cbase + CE)
            cnt = plsc.all_reduce_population_count(m)[0]

            @pl.when(cnt > 0)
            def _fire():
                slot = lax.rem(f, RING)
                sb = slot * STG
                @pl.when(f >= RING - 1)
                def _drain():
                    pltpu.make_async_copy(
                        hstage.at[pl.ds(0, STG)], stgh.at[pl.ds(0, STG)],
                        sem).wait()
                    pltpu.make_async_copy(
                        tstage.at[pl.ds(0, STG)], stgt.at[pl.ds(0, STG)],
                        sem).wait()
                rank = plsc.cumsum(m.astype(jnp.int32)) - 1
                coltmp[...] = jnp.zeros((LANES,), jnp.int32)
                jtmp[...] = NX + wid * LANES + iota
                plsc.store_scatter(coltmp, [rank], xoff - cbase + coladd,
                                   mask=m)
                plsc.store_scatter(jtmp, [rank], jv16, mask=m)
                ctv = coltmp[...]
                jv = jtmp[...]
                for d in range(DIM):
                    dsp = jnp.full((LANES,), d, jnp.int32)
                    hv = plsc.load_gather(hch, [dsp, ctv])
                    tv = plsc.load_gather(tch, [dsp, ctv])
                    plsc.store_scatter(stgh, [sb + iota * DIM + d], hv)
                    plsc.store_scatter(stgt, [sb + iota * DIM + d], tv)
                for k in range(LANES):
                    jk = jv[k]
                    pltpu.async_copy(stgh.at[pl.ds(sb + k * DIM, DIM)],
                                     hstage.at[pl.ds(jk * DIM, DIM)], sem)
                    pltpu.async_copy(stgt.at[pl.ds(sb + k * DIM, DIM)],
                                     tstage.at[pl.ds(jk * DIM, DIM)], sem)

            fn = f + jnp.where(cnt > 0, 1, 0)
            all_below = (xv[LANES - 1] - elo) < cbase + CE
            pn = jnp.minimum(P + 1, NV - 1)
            go = all_below & (pn != P)
            return (jnp.where(all_below, pn, P), fn, go)

        P, f, _ = lax.while_loop(wcond, wbody, (P, f, jnp.bool_(True)))
        return P, f

    def fire_chunk(ci):
        par = lax.rem(ci, NBUF) * CE

        @pl.when(ci < nreg)
        def _reg():
            pltpu.async_copy(
                eht.at[:, pl.ds(elo + ci * CE, CE)],
                hch.at[:, pl.ds(par, CE)], sem2)
            pltpu.async_copy(
                ett.at[:, pl.ds(elo + ci * CE, CE)],
                tch.at[:, pl.ds(par, CE)], sem2)

        @pl.when(is_last & (ci == nreg))
        def _tail():
            pltpu.async_copy(lasth, hch.at[:, pl.ds(par, 128)], sem2)
            pltpu.async_copy(lastt, tch.at[:, pl.ds(par, 128)], sem2)

    for pre in range(NBUF - 1):
        fire_chunk(pre)

    pltpu.make_async_copy(xs_s, xsv, sem3).wait()
    pltpu.make_async_copy(order, ordv, sem3).wait()
    p0, _ = lax.fori_loop(0, 12, bis, (jnp.int32(0), jnp.int32(NV)))

    def chunk_fn(ci, st):
        P, f = st
        par = lax.rem(ci, NBUF) * CE
        is_tail = is_last & (ci == nreg)

        @pl.when(jnp.logical_not(is_tail))
        def _dr():
            pltpu.make_async_copy(eht.at[:, pl.ds(0, CE)],
                                  hch.at[:, pl.ds(par, CE)], sem2).wait()
            pltpu.make_async_copy(eht.at[:, pl.ds(0, CE)],
                                  tch.at[:, pl.ds(par, CE)], sem2).wait()

        @pl.when(is_tail)
        def _drt():
            pltpu.make_async_copy(eht.at[:, pl.ds(0, 128)],
                                  hch.at[:, pl.ds(par, 128)], sem2).wait()
            pltpu.make_async_copy(eht.at[:, pl.ds(0, 128)],
                                  tch.at[:, pl.ds(par, 128)], sem2).wait()

        fire_chunk(ci + NBUF - 1)
        coladd = par + jnp.where(is_tail, 64, 0)
        return scan_chunk(ci, coladd, P, f)

    _, f = lax.fori_loop(0, nch, chunk_fn, (p0, jnp.int32(0)))

    for k in range(RING - 1):
        @pl.when(f > k)
        def _final_drain():
            pltpu.make_async_copy(
                hstage.at[pl.ds(0, STG)], stgh.at[pl.ds(0, STG)], sem).wait()
            pltpu.make_async_copy(
                tstage.at[pl.ds(0, STG)], stgt.at[pl.ds(0, STG)], sem).wait()


def _combine_body(rels, hstage, tstage, rf, ri, out,
                  ridx, h1b, t1b, h2b, t2b, r1b, r2b, qt, outv, sem):
    c = lax.axis_index("c")
    s = lax.axis_index("s")
    wid = s * NUM_CORES + c
    base = wid * (BATCH // NW)
    sz = SUB * DIM

    pltpu.sync_copy(rels.at[pl.ds(base, BATCH // NW)], ridx)

    iota = lax.iota(jnp.int32, LANES)

    for sb in range(NSUB):
        s0 = base + sb * SUB
        off = sb * sz
        pltpu.async_copy(hstage.at[pl.ds(s0 * DIM, sz)],
                         h1b.at[pl.ds(off, sz)], sem)
        pltpu.async_copy(tstage.at[pl.ds(s0 * DIM, sz)],
                         t2b.at[pl.ds(off, sz)], sem)
        pltpu.async_copy(hstage.at[pl.ds((s0 + BATCH) * DIM, sz)],
                         h2b.at[pl.ds(off, sz)], sem)
        pltpu.async_copy(tstage.at[pl.ds((s0 + BATCH) * DIM, sz)],
                         t1b.at[pl.ds(off, sz)], sem)
        rsl = ridx.at[pl.ds(sb * SUB, SUB)]
        pltpu.async_copy(rf.at[rsl], r1b.at[pl.ds(sb * SUB, SUB)], sem)
        pltpu.async_copy(ri.at[rsl], r2b.at[pl.ds(sb * SUB, SUB)], sem)

    def sub_fn(sb, s0, off):
        pltpu.make_async_copy(hstage.at[pl.ds(0, sz)],
                              h1b.at[pl.ds(off, sz)], sem).wait()
        pltpu.make_async_copy(hstage.at[pl.ds(0, sz)],
                              t2b.at[pl.ds(off, sz)], sem).wait()
        pltpu.make_async_copy(hstage.at[pl.ds(0, sz)],
                              h2b.at[pl.ds(off, sz)], sem).wait()
        pltpu.make_async_copy(hstage.at[pl.ds(0, sz)],
                              t1b.at[pl.ds(off, sz)], sem).wait()
        pltpu.make_async_copy(rf.at[pl.ds(0, SUB)],
                              r1b.at[pl.ds(sb * SUB, SUB)], sem).wait()
        pltpu.make_async_copy(rf.at[pl.ds(0, SUB)],
                              r2b.at[pl.ds(sb * SUB, SUB)], sem).wait()
        def row_fn(i, carry):
            lo = pl.ds(i * DIM, LANES)
            hi = pl.ds(i * DIM + LANES, LANES)
            rlo = pl.ds(0, LANES)
            rhi = pl.ds(LANES, LANES)
            pa = (h1b[lo] * r1b[i, rlo] * t1b[lo]
                  + h2b[lo] * r2b[i, rlo] * t2b[lo])
            pb = (h1b[hi] * r1b[i, rhi] * t1b[hi]
                  + h2b[hi] * r2b[i, rhi] * t2b[hi])
            q = pa + pb
            g = i // LANES
            l = i - g * LANES
            flat = g * (LANES * LANES) + iota * LANES + l
            plsc.store_scatter(qt, [flat], q)
            return carry

        lax.fori_loop(sb * SUB, (sb + 1) * SUB, row_fn, 0)

    for sb in range(NSUB):
        sub_fn(sb, base + sb * SUB, sb * sz)

    def grp_fn(g, carry):
        gb = g * (LANES * LANES)
        acc = qt[pl.ds(gb, LANES)]
        for dd in range(1, LANES):
            acc = acc + qt[pl.ds(gb + dd * LANES, LANES)]
        outv[pl.ds(g * LANES, LANES)] = acc * 0.5
        return carry

    lax.fori_loop(0, (BATCH // NW) // LANES, grp_fn, 0)

    pltpu.sync_copy(outv, out.at[pl.ds(base, BATCH // NW)])


@jax.jit
def kernel(heads, rels, tails, ent_embs_h, ent_embs_t, rel_embs_f, rel_embs_i):
    heads = heads.astype(jnp.int32)
    rels = rels.astype(jnp.int32)
    tails = tails.astype(jnp.int32)

    xs = jnp.concatenate([heads, tails])
    xs_s, order = lax.sort((xs, jnp.arange(NX, dtype=jnp.int32)), num_keys=1)
    eht = ent_embs_h.T
    ett = ent_embs_t.T
    lasth = lax.slice(eht, (0, TAILBUF_BASE), (DIM, NUM_E))
    lastt = lax.slice(ett, (0, TAILBUF_BASE), (DIM, NUM_E))

    mesh = plsc.VectorSubcoreMesh(
        core_axis_name="c", subcore_axis_name="s",
        num_cores=NUM_CORES, num_subcores=NUM_SUBCORES)

    extract = pl.kernel(
        _extract_body,
        out_type=(
            jax.ShapeDtypeStruct(((NX + NDUMP) * DIM,), jnp.float32),
            jax.ShapeDtypeStruct(((NX + NDUMP) * DIM,), jnp.float32),
        ),
        mesh=mesh,
        scratch_types=[
            pltpu.VMEM((NX,), jnp.int32),
            pltpu.VMEM((NX,), jnp.int32),
            pltpu.VMEM((DIM, NBUF * CE), jnp.float32),
            pltpu.VMEM((DIM, NBUF * CE), jnp.float32),
            pltpu.VMEM((RING * STG,), jnp.float32),
            pltpu.VMEM((RING * STG,), jnp.float32),
            pltpu.VMEM((LANES,), jnp.int32),
            pltpu.VMEM((LANES,), jnp.int32),
            pltpu.SemaphoreType.DMA,
            pltpu.SemaphoreType.DMA,
            pltpu.SemaphoreType.DMA,
        ],
        compiler_params=pltpu.CompilerParams(needs_layout_passes=False),
        name="simple_extract_sc",
    )
    hstage, tstage = extract(xs_s, order, eht, ett, lasth, lastt)

    combine = pl.kernel(
        _combine_body,
        out_type=jax.ShapeDtypeStruct((BATCH,), jnp.float32),
        mesh=mesh,
        scratch_types=[
            pltpu.VMEM((BATCH // NW,), jnp.int32),
            pltpu.VMEM((NSUB * SUB * DIM,), jnp.float32),
            pltpu.VMEM((NSUB * SUB * DIM,), jnp.float32),
            pltpu.VMEM((NSUB * SUB * DIM,), jnp.float32),
            pltpu.VMEM((NSUB * SUB * DIM,), jnp.float32),
            pltpu.VMEM((NSUB * SUB, DIM), jnp.float32),
            pltpu.VMEM((NSUB * SUB, DIM), jnp.float32),
            pltpu.VMEM(((BATCH // NW) * LANES,), jnp.float32),
            pltpu.VMEM((BATCH // NW,), jnp.float32),
            pltpu.SemaphoreType.DMA,
        ],
        compiler_params=pltpu.CompilerParams(
            needs_layout_passes=False, use_tc_tiling_on_sc=False),
        name="simple_combine_sc",
    )
    return combine(rels, hstage, tstage, rel_embs_f, rel_embs_i)
